# baseline (speedup 1.0000x reference)
import functools

import jax
import jax.numpy as jnp
from jax import lax
from jax.experimental import pallas as pl
from jax.experimental.pallas import tpu as pltpu
from jax.experimental.pallas import tpu_sc as plsc

NW = 32
CB = 2
LANES = 16


NBUF = 4


def _emb_mean_kernel(B, L, D, idx_hbm, table_hbm, out_hbm,
                     idx_v, rows0, rows1, rows2, rows3, acc_v,
                     sem0, sem1, sem2, sem3):
    BPW = B // NW
    NCH = BPW // CB
    ROWS = CB * L
    NV = D // LANES
    inv = jnp.float32(1.0 / L)

    nc = plsc.get_sparse_core_info().num_cores
    wid = lax.axis_index("s") * nc + lax.axis_index("c")

    pltpu.sync_copy(idx_hbm.at[wid], idx_v)

    bufs = (rows0, rows1, rows2, rows3)
    sems = (sem0, sem1, sem2, sem3)

    def start(c, b):
        pltpu.async_copy(table_hbm.at[idx_v.at[c]], bufs[b], sems[b])

    def wait(b):
        pltpu.make_async_copy(table_hbm.at[idx_v.at[0]], bufs[b], sems[b]).wait()

    def accumulate(c, b):
        rows = bufs[b]
        for j in range(CB):
            regs = [jnp.zeros((LANES,), jnp.float32) for _ in range(NV)]
            for r in range(L):
                for k in range(NV):
                    regs[k] = regs[k] + rows[j * L + r, pl.ds(k * LANES, LANES)]
            for k in range(NV):
                acc_v[c * CB + j, pl.ds(k * LANES, LANES)] = regs[k] * inv

    for b in range(NBUF):
        start(b, b)

    def body(g, carry):
        for b in range(NBUF):
            c = NBUF * g + b
            wait(b)
            accumulate(c, b)

            @pl.when(c + NBUF < NCH)
            def _():
                start(c + NBUF, b)
        return carry

    lax.fori_loop(0, NCH // NBUF, body, 0)

    pltpu.sync_copy(acc_v, out_hbm.at[pl.ds(wid * BPW, BPW)])


@functools.partial(jax.jit, static_argnames=("B", "L", "D"))
def _emb_mean(idx, W, B, L, D):
    BPW = B // NW
    NCH = BPW // CB
    ROWS = CB * L
    mesh = plsc.VectorSubcoreMesh(core_axis_name="c", subcore_axis_name="s")
    return pl.kernel(
        functools.partial(_emb_mean_kernel, B, L, D),
        out_type=jax.ShapeDtypeStruct((B, D), jnp.float32),
        mesh=mesh,
        compiler_params=pltpu.CompilerParams(use_tc_tiling_on_sc=False),
        scratch_types=[
            pltpu.VMEM((NCH, ROWS), jnp.int32),
            pltpu.VMEM((ROWS, D), jnp.float32),
            pltpu.VMEM((ROWS, D), jnp.float32),
            pltpu.VMEM((ROWS, D), jnp.float32),
            pltpu.VMEM((ROWS, D), jnp.float32),
            pltpu.VMEM((BPW, D), jnp.float32),
            pltpu.SemaphoreType.DMA,
            pltpu.SemaphoreType.DMA,
            pltpu.SemaphoreType.DMA,
            pltpu.SemaphoreType.DMA,
        ],
    )(idx, W)


def kernel(word_ids, W):
    B, L = word_ids.shape
    D = W.shape[1]
    BPW = B // NW
    assert B % NW == 0 and BPW % CB == 0 and D % LANES == 0
    idx = word_ids.astype(jnp.int32).reshape(NW, BPW // CB, CB * L)
    return _emb_mean(idx, W, B, L, D)


# speedup vs baseline: 1.0623x; 1.0623x over previous
import functools

import jax
import jax.numpy as jnp
from jax import lax
from jax.experimental import pallas as pl
from jax.experimental.pallas import tpu as pltpu
from jax.experimental.pallas import tpu_sc as plsc

NW = 32
LANES = 16
NBUF = 4


def _emb_mean_kernel(B, L, D, idx_hbm, table_hbm, out_hbm,
                     idx_v, rows0, rows1, rows2, rows3, acc_v,
                     sem0, sem1, sem2, sem3):
    BPW = B // NW
    NV = D // LANES
    inv = jnp.float32(1.0 / L)

    nc = plsc.get_sparse_core_info().num_cores
    wid = lax.axis_index("s") * nc + lax.axis_index("c")

    pltpu.sync_copy(idx_hbm.at[pl.ds(wid * BPW, BPW)], idx_v)

    bufs = (rows0, rows1, rows2, rows3)
    sems = (sem0, sem1, sem2, sem3)

    def start(c, b):
        pltpu.async_copy(table_hbm.at[idx_v.at[c]], bufs[b], sems[b])

    def wait(b):
        pltpu.make_async_copy(table_hbm.at[idx_v.at[0]], bufs[b], sems[b]).wait()

    def accumulate(c, b):
        rows = bufs[b]
        regs = [jnp.zeros((LANES,), jnp.float32) for _ in range(NV)]
        for r in range(L):
            for k in range(NV):
                regs[k] = regs[k] + rows[r, pl.ds(k * LANES, LANES)]
        for k in range(NV):
            acc_v[c, pl.ds(k * LANES, LANES)] = regs[k] * inv

    for b in range(NBUF):
        start(b, b)

    def body(g, carry):
        for b in range(NBUF):
            c = NBUF * g + b
            wait(b)
            accumulate(c, b)

            @pl.when(c + NBUF < BPW)
            def _():
                start(c + NBUF, b)
        return carry

    lax.fori_loop(0, BPW // NBUF, body, 0)

    pltpu.sync_copy(acc_v, out_hbm.at[pl.ds(wid * BPW, BPW)])


@functools.partial(jax.jit, static_argnames=("B", "L", "D"))
def _emb_mean(idx, W, B, L, D):
    BPW = B // NW
    mesh = plsc.VectorSubcoreMesh(core_axis_name="c", subcore_axis_name="s")
    return pl.kernel(
        functools.partial(_emb_mean_kernel, B, L, D),
        out_type=jax.ShapeDtypeStruct((B, D), jnp.float32),
        mesh=mesh,
        compiler_params=pltpu.CompilerParams(use_tc_tiling_on_sc=False),
        scratch_types=[
            pltpu.VMEM((BPW, L), jnp.int32),
            pltpu.VMEM((L, D), jnp.float32),
            pltpu.VMEM((L, D), jnp.float32),
            pltpu.VMEM((L, D), jnp.float32),
            pltpu.VMEM((L, D), jnp.float32),
            pltpu.VMEM((BPW, D), jnp.float32),
            pltpu.SemaphoreType.DMA,
            pltpu.SemaphoreType.DMA,
            pltpu.SemaphoreType.DMA,
            pltpu.SemaphoreType.DMA,
        ],
    )(idx, W)


def kernel(word_ids, W):
    B, L = word_ids.shape
    D = W.shape[1]
    assert B % NW == 0 and D % LANES == 0
    return _emb_mean(word_ids.astype(jnp.int32), W, B, L, D)


# speedup vs baseline: 1.0815x; 1.0181x over previous
import functools

import jax
import jax.numpy as jnp
from jax import lax
from jax.experimental import pallas as pl
from jax.experimental.pallas import tpu as pltpu
from jax.experimental.pallas import tpu_sc as plsc

NW = 32
LANES = 16
NBUF = 4
SPG = 128


def _emb_mean_kernel(B, L, D, idx_hbm, table_hbm, out_hbm,
                     idx_v, rows0, rows1, rows2, rows3, acc_v,
                     sem0, sem1, sem2, sem3):
    BPW = B // NW
    NBLK = BPW // SPG
    NV = D // LANES
    NS = L * NBLK
    inv = jnp.float32(1.0 / L)

    nc = plsc.get_sparse_core_info().num_cores
    wid = lax.axis_index("s") * nc + lax.axis_index("c")

    pltpu.sync_copy(idx_hbm.at[:, pl.ds(wid * BPW, BPW)], idx_v)

    def zbody(r, carry):
        for k in range(NV):
            acc_v[r, pl.ds(k * LANES, LANES)] = jnp.zeros((LANES,), jnp.float32)
        return carry

    lax.fori_loop(0, BPW, zbody, 0)

    bufs = (rows0, rows1, rows2, rows3)
    sems = (sem0, sem1, sem2, sem3)

    def start(s, b):
        l = s // NBLK
        blk = s - l * NBLK
        idx_slice = idx_v.at[l, pl.ds(blk * SPG, SPG)]
        pltpu.async_copy(table_hbm.at[idx_slice], bufs[b], sems[b])

    def wait(b):
        pltpu.make_async_copy(
            table_hbm.at[idx_v.at[0, pl.ds(0, SPG)]], bufs[b], sems[b]
        ).wait()

    def accumulate(s, b):
        blk = s - (s // NBLK) * NBLK
        base = blk * SPG
        rows = bufs[b]

        def abody(r, carry):
            for k in range(NV):
                sl = pl.ds(k * LANES, LANES)
                plsc.addupdate(acc_v.at[base + r, sl], rows[r, sl])
            return carry

        lax.fori_loop(0, SPG, abody, 0)

    for b in range(NBUF):
        start(b, b)

    def body(g, carry):
        for b in range(NBUF):
            s = NBUF * g + b
            wait(b)
            accumulate(s, b)

            @pl.when(s + NBUF < NS)
            def _():
                start(s + NBUF, b)
        return carry

    lax.fori_loop(0, NS // NBUF, body, 0)

    def sbody(r, carry):
        for k in range(NV):
            sl = pl.ds(k * LANES, LANES)
            acc_v[r, sl] = acc_v[r, sl] * inv
        return carry

    lax.fori_loop(0, BPW, sbody, 0)
    pltpu.sync_copy(acc_v, out_hbm.at[pl.ds(wid * BPW, BPW)])


@functools.partial(jax.jit, static_argnames=("B", "L", "D"))
def _emb_mean(idx_t, W, B, L, D):
    BPW = B // NW
    mesh = plsc.VectorSubcoreMesh(core_axis_name="c", subcore_axis_name="s")
    return pl.kernel(
        functools.partial(_emb_mean_kernel, B, L, D),
        out_type=jax.ShapeDtypeStruct((B, D), jnp.float32),
        mesh=mesh,
        compiler_params=pltpu.CompilerParams(use_tc_tiling_on_sc=False),
        scratch_types=[
            pltpu.VMEM((L, BPW), jnp.int32),
            pltpu.VMEM((SPG, D), jnp.float32),
            pltpu.VMEM((SPG, D), jnp.float32),
            pltpu.VMEM((SPG, D), jnp.float32),
            pltpu.VMEM((SPG, D), jnp.float32),
            pltpu.VMEM((BPW, D), jnp.float32),
            pltpu.SemaphoreType.DMA,
            pltpu.SemaphoreType.DMA,
            pltpu.SemaphoreType.DMA,
            pltpu.SemaphoreType.DMA,
        ],
    )(idx_t, W)


def kernel(word_ids, W):
    B, L = word_ids.shape
    D = W.shape[1]
    BPW = B // NW
    assert B % NW == 0 and BPW % SPG == 0 and D % LANES == 0
    idx_t = word_ids.astype(jnp.int32).T
    return _emb_mean(idx_t, W, B, L, D)
